# baseline (speedup 1.0000x reference)
import functools

import jax
import jax.numpy as jnp
from jax.experimental import pallas as pl
from jax.experimental.pallas import tpu as pltpu

N = 32768
IN_FEAT = 768
OUT_FEAT = 768
BLOCK_ROWS = 1024


def _masked_linear_block(x_ref, m_ref, w_ref, b_ref, o_ref):
    xb = x_ref[...].astype(jnp.bfloat16)
    y = jax.lax.dot_general(
        xb,
        w_ref[...],
        (((1,), (1,)), ((), ())),
        preferred_element_type=jnp.float32,
    )
    y = y + b_ref[...]
    o_ref[...] = jnp.where(m_ref[...] != 0, y, 0.0)


@jax.jit
def _masked_linear(x, amask2d, w_bf16, b2d):
    grid = (N // BLOCK_ROWS,)
    return pl.pallas_call(
        _masked_linear_block,
        grid=grid,
        in_specs=[
            pl.BlockSpec((BLOCK_ROWS, IN_FEAT), lambda i: (i, 0)),
            pl.BlockSpec((BLOCK_ROWS, 1), lambda i: (i, 0)),
            pl.BlockSpec((OUT_FEAT, IN_FEAT), lambda i: (0, 0)),
            pl.BlockSpec((1, OUT_FEAT), lambda i: (0, 0)),
        ],
        out_specs=pl.BlockSpec((BLOCK_ROWS, OUT_FEAT), lambda i: (i, 0)),
        out_shape=jax.ShapeDtypeStruct((N, OUT_FEAT), jnp.float32),
        compiler_params=pltpu.CompilerParams(
            dimension_semantics=("arbitrary",),
        ),
    )(x, amask2d, w_bf16, b2d)


def kernel(x, amask, W, b):
    return _masked_linear(
        x,
        amask.reshape(N, 1),
        W.astype(jnp.bfloat16),
        b.reshape(1, OUT_FEAT),
    )


# speedup vs baseline: 1.0124x; 1.0124x over previous
import functools

import jax
import jax.numpy as jnp
from jax.experimental import pallas as pl
from jax.experimental.pallas import tpu as pltpu

N = 32768
IN_FEAT = 768
OUT_FEAT = 768
BLOCK_ROWS = 1024


def _masked_linear_block(x_ref, m_ref, w_ref, b_ref, o_ref):
    xb = x_ref[...].astype(jnp.bfloat16)
    y = jax.lax.dot_general(
        xb,
        w_ref[...],
        (((1,), (0,)), ((), ())),
        preferred_element_type=jnp.float32,
    )
    y = y + b_ref[...]
    o_ref[...] = jnp.where(m_ref[...] != 0, y, 0.0)


@jax.jit
def _masked_linear(x, amask2d, w_bf16, b2d):
    grid = (N // BLOCK_ROWS,)
    return pl.pallas_call(
        _masked_linear_block,
        grid=grid,
        in_specs=[
            pl.BlockSpec((BLOCK_ROWS, IN_FEAT), lambda i: (i, 0)),
            pl.BlockSpec((BLOCK_ROWS, 1), lambda i: (i, 0)),
            pl.BlockSpec((IN_FEAT, OUT_FEAT), lambda i: (0, 0)),
            pl.BlockSpec((1, OUT_FEAT), lambda i: (0, 0)),
        ],
        out_specs=pl.BlockSpec((BLOCK_ROWS, OUT_FEAT), lambda i: (i, 0)),
        out_shape=jax.ShapeDtypeStruct((N, OUT_FEAT), jnp.float32),
        compiler_params=pltpu.CompilerParams(
            dimension_semantics=("arbitrary",),
        ),
    )(x, amask2d, w_bf16, b2d)


def kernel(x, amask, W, b):
    return _masked_linear(
        x,
        amask.reshape(N, 1),
        W.T.astype(jnp.bfloat16),
        b.reshape(1, OUT_FEAT),
    )
